# Initial kernel scaffold; baseline (speedup 1.0000x reference)
import jax
import jax.numpy as jnp
from jax.experimental import pallas as pl


def kernel(p_bin, p_inst, p_dir, binary_mask, instance_masks, direction_field):
    raise NotImplementedError("write your pallas kernel here")


# speedup vs baseline: 18.3135x; 18.3135x over previous
import jax
import jax.numpy as jnp
from jax import lax
from jax.experimental import pallas as pl
from jax.experimental.pallas import tpu as pltpu

_B, _H, _W = 4, 512, 512
_N = _H * _W
_K = max(1, int(_N * 0.1))
_BH = 128
_NH = _H // _BH
_NPASS = 8

_INTERPRET = False


def _fused_body(pbin_ref, pinst_ref, pdir_ref, bmask_ref, imask_ref,
                dfield_ref, loss_ref, sums_ref, acc_ref):
    b = pl.program_id(0)
    h = pl.program_id(1)

    x = pbin_ref[0, 0]
    t = bmask_ref[0, 0]
    bce = jnp.maximum(x, 0.0) - x * t + jnp.log(1.0 + jnp.exp(-jnp.abs(x)))
    loss_ref[0] = bce

    mx = pinst_ref[0, 0]
    for c in range(1, 10):
        mx = jnp.maximum(mx, pinst_ref[0, c])
    sexp = jnp.exp(pinst_ref[0, 0] - mx)
    for c in range(1, 10):
        sexp = sexp + jnp.exp(pinst_ref[0, c] - mx)
    lse = mx + jnp.log(sexp)
    best_m = imask_ref[0, 0]
    best_logit = pinst_ref[0, 0]
    for c in range(1, 10):
        m = imask_ref[0, c]
        upd = m > best_m
        best_m = jnp.where(upd, m, best_m)
        best_logit = jnp.where(upd, pinst_ref[0, c], best_logit)
    ce = lse - best_logit
    valid = (t > 0.5).astype(jnp.float32)
    ce_part = jnp.sum(ce * valid)
    val_part = jnp.sum(valid)

    d0 = dfield_ref[0, 0]
    d1 = dfield_ref[0, 1]
    nrm = jnp.sqrt(d0 * d0 + d1 * d1)
    inv = 1.0 / jnp.maximum(nrm, 1e-6)
    dt0 = d0 * inv
    dt1 = d1 * inv
    dvalid = (nrm * inv > 1e-6).astype(jnp.float32)
    e0 = pdir_ref[0, 0] - dt0
    e1 = pdir_ref[0, 1] - dt1
    dir_part = jnp.sum((e0 * e0 + e1 * e1) * dvalid)
    dval_part = jnp.sum(dvalid)

    first = jnp.logical_and(b == 0, h == 0)

    @pl.when(first)
    def _init():
        acc_ref[0] = ce_part
        acc_ref[1] = val_part
        acc_ref[2] = dir_part
        acc_ref[3] = dval_part

    @pl.when(jnp.logical_not(first))
    def _acc():
        acc_ref[0] = acc_ref[0] + ce_part
        acc_ref[1] = acc_ref[1] + val_part
        acc_ref[2] = acc_ref[2] + dir_part
        acc_ref[3] = acc_ref[3] + dval_part

    last = jnp.logical_and(b == _B - 1, h == _NH - 1)

    @pl.when(last)
    def _emit():
        sums_ref[0, 0] = acc_ref[0]
        sums_ref[0, 1] = acc_ref[1]
        sums_ref[0, 2] = acc_ref[2]
        sums_ref[0, 3] = acc_ref[3]


def _bits_to_f32(bits_scalar):
    vecb = jnp.full((8, 128), bits_scalar, dtype=jnp.int32)
    vecf = lax.bitcast_convert_type(vecb, jnp.float32)
    r0 = lax.broadcasted_iota(jnp.int32, (8, 128), 0) == 0
    c0 = lax.broadcasted_iota(jnp.int32, (8, 128), 1) == 0
    return jnp.sum(jnp.where(jnp.logical_and(r0, c0), vecf, 0.0))


def _select_body(loss_ref, sums_ref, out_ref):
    kf = jnp.float32(_K)
    lbin_sum = jnp.float32(0.0)
    for b in range(_B):
        v = loss_ref[b]
        bits = lax.bitcast_convert_type(v, jnp.int32)

        def radix_pass(p, carry):
            prefix, remk, cab = carry
            shift = jnp.maximum(27 - 4 * p, 0)
            j_star = jnp.int32(0)
            within_next = jnp.float32(0.0)
            for j in range(1, 16):
                tj = prefix + lax.shift_left(jnp.int32(j), shift)
                cj = jnp.sum((bits >= tj).astype(jnp.float32))
                within_j = cj - cab
                take = within_j >= remk
                j_star = j_star + jnp.where(take, 1, 0).astype(jnp.int32)
                within_next = jnp.where(
                    take, within_next, jnp.maximum(within_next, within_j))
            prefix = prefix + lax.shift_left(j_star, shift)
            cab = cab + within_next
            remk = remk - within_next
            return prefix, remk, cab

        prefix, remk, cab = lax.fori_loop(
            0, _NPASS, radix_pass,
            (jnp.int32(0), kf, jnp.float32(0.0)))

        cgt = jnp.sum((bits > prefix).astype(jnp.float32))
        sgt = jnp.sum(jnp.where(bits > prefix, v, 0.0))
        tval = _bits_to_f32(prefix)
        lbin_sum = lbin_sum + sgt + (kf - cgt) * tval

    l_bin = lbin_sum / jnp.float32(_B * _K)
    l_inst = sums_ref[0, 0] / jnp.maximum(sums_ref[0, 1], 1.0)
    l_dir = sums_ref[0, 2] / jnp.maximum(sums_ref[0, 3], 1.0)
    out_ref[0, 0] = l_bin + l_inst + l_dir


def kernel(p_bin, p_inst, p_dir, binary_mask, instance_masks, direction_field):
    loss_map, sums = pl.pallas_call(
        _fused_body,
        grid=(_B, _NH),
        in_specs=[
            pl.BlockSpec((1, 1, _BH, _W), lambda b, h: (b, 0, h, 0)),
            pl.BlockSpec((1, 10, _BH, _W), lambda b, h: (b, 0, h, 0)),
            pl.BlockSpec((1, 2, _BH, _W), lambda b, h: (b, 0, h, 0)),
            pl.BlockSpec((1, 1, _BH, _W), lambda b, h: (b, 0, h, 0)),
            pl.BlockSpec((1, 10, _BH, _W), lambda b, h: (b, 0, h, 0)),
            pl.BlockSpec((1, 2, _BH, _W), lambda b, h: (b, 0, h, 0)),
        ],
        out_specs=[
            pl.BlockSpec((1, _BH, _W), lambda b, h: (b, h, 0)),
            pl.BlockSpec(memory_space=pltpu.SMEM),
        ],
        out_shape=[
            jax.ShapeDtypeStruct((_B, _H, _W), jnp.float32),
            jax.ShapeDtypeStruct((1, 4), jnp.float32),
        ],
        scratch_shapes=[pltpu.SMEM((4,), jnp.float32)],
        interpret=_INTERPRET,
    )(p_bin, p_inst, p_dir, binary_mask, instance_masks, direction_field)

    total = pl.pallas_call(
        _select_body,
        in_specs=[
            pl.BlockSpec(memory_space=pltpu.VMEM),
            pl.BlockSpec(memory_space=pltpu.SMEM),
        ],
        out_specs=pl.BlockSpec(memory_space=pltpu.SMEM),
        out_shape=jax.ShapeDtypeStruct((1, 1), jnp.float32),
        interpret=_INTERPRET,
    )(loss_map, sums)

    return total[0, 0]


# speedup vs baseline: 21.4098x; 1.1691x over previous
import functools

import jax
import jax.numpy as jnp
from jax import lax
from jax.experimental import pallas as pl
from jax.experimental.pallas import tpu as pltpu
from jax.experimental.pallas import tpu_sc as plsc

_B, _H, _W = 4, 512, 512
_N = _H * _W
_K = max(1, int(_N * 0.1))
_BH = 128
_NH = _H // _BH

_CHUNK = _N // 16

_INTERPRET = False


def _bce_body(pbin_ref, bmask_ref, loss_ref):
    x = pbin_ref[0, 0]
    t = bmask_ref[0, 0]
    loss_ref[0] = (jnp.maximum(x, 0.0) - x * t
                   + jnp.log(1.0 + jnp.exp(-jnp.abs(x))))


def _sums_body(pinst_ref, pdir_ref, bmask_ref, imask_ref, dfield_ref,
               sums_ref, acc_ref):
    b = pl.program_id(0)
    h = pl.program_id(1)

    mx = pinst_ref[0, 0]
    for c in range(1, 10):
        mx = jnp.maximum(mx, pinst_ref[0, c])
    sexp = jnp.exp(pinst_ref[0, 0] - mx)
    for c in range(1, 10):
        sexp = sexp + jnp.exp(pinst_ref[0, c] - mx)
    lse = mx + jnp.log(sexp)
    best_m = imask_ref[0, 0]
    best_logit = pinst_ref[0, 0]
    for c in range(1, 10):
        m = imask_ref[0, c]
        upd = m > best_m
        best_m = jnp.where(upd, m, best_m)
        best_logit = jnp.where(upd, pinst_ref[0, c], best_logit)
    ce = lse - best_logit
    valid = (bmask_ref[0, 0] > 0.5).astype(jnp.float32)
    ce_part = jnp.sum(ce * valid)
    val_part = jnp.sum(valid)

    d0 = dfield_ref[0, 0]
    d1 = dfield_ref[0, 1]
    nrm = jnp.sqrt(d0 * d0 + d1 * d1)
    inv = 1.0 / jnp.maximum(nrm, 1e-6)
    dt0 = d0 * inv
    dt1 = d1 * inv
    dvalid = (nrm * inv > 1e-6).astype(jnp.float32)
    e0 = pdir_ref[0, 0] - dt0
    e1 = pdir_ref[0, 1] - dt1
    dir_part = jnp.sum((e0 * e0 + e1 * e1) * dvalid)
    dval_part = jnp.sum(dvalid)

    first = jnp.logical_and(b == 0, h == 0)

    @pl.when(first)
    def _init():
        acc_ref[0] = ce_part
        acc_ref[1] = val_part
        acc_ref[2] = dir_part
        acc_ref[3] = dval_part

    @pl.when(jnp.logical_not(first))
    def _acc():
        acc_ref[0] = acc_ref[0] + ce_part
        acc_ref[1] = acc_ref[1] + val_part
        acc_ref[2] = acc_ref[2] + dir_part
        acc_ref[3] = acc_ref[3] + dval_part

    last = jnp.logical_and(b == _B - 1, h == _NH - 1)

    @pl.when(last)
    def _emit():
        sums_ref[0, 0] = acc_ref[0]
        sums_ref[0, 1] = acc_ref[1]
        sums_ref[0, 2] = acc_ref[2]
        sums_ref[0, 3] = acc_ref[3]


def _scan_hist(merged_v, r, remk_vec, iota16):
    def sbody(i, carry):
        acc, cnt = carry
        v = 31 - i
        block = merged_v[pl.ds(r * 512 + v * 16, 16)]
        csum = plsc.cumsum(lax.rev(block, (0,))) + acc
        mask = csum >= remk_vec
        cnt = cnt + plsc.all_reduce_population_count(mask)
        acc = acc + jnp.sum(block)
        return acc, cnt

    _, cnt = lax.fori_loop(
        0, 32, sbody, (jnp.float32(0.0), jnp.zeros((16,), jnp.int32)))
    bstar = cnt - 1

    def abody(i, a):
        v = 31 - i
        block = merged_v[pl.ds(r * 512 + v * 16, 16)]
        binid = v * 16 + iota16
        return a + jnp.sum(jnp.where(binid > bstar, block, 0.0))

    cab = lax.fori_loop(0, 32, abody, jnp.float32(0.0))
    return bstar, cab


def _sc_select(loss_hbm, out_hbm, data_v, hist_v, big_v, merged_v,
               ctrl_v, misc_v, outbuf_v, sp_hists, sp_ctrl):
    c = lax.axis_index("c")
    s = lax.axis_index("s")
    ones16 = jnp.ones((16,), jnp.float32)
    zero16f = jnp.zeros((16,), jnp.float32)
    iota16 = lax.iota(jnp.int32, 16)
    kvec = ones16 * jnp.float32(_K)

    for r in range(2):
        pltpu.sync_copy(
            loss_hbm.at[c * 2 + r, pl.ds(s * _CHUNK, _CHUNK)],
            data_v.at[pl.ds(r * _CHUNK, _CHUNK)])

    def zbody(i, _):
        hist_v[pl.ds(i * 16, 16)] = zero16f
        return 0

    lax.fori_loop(0, 64, zbody, 0)

    def h1(i, _):
        for r in range(2):
            x = data_v[pl.ds(r * _CHUNK + i * 16, 16)]
            bits = plsc.bitcast(x, jnp.int32)
            hi = lax.shift_right_logical(bits, 22)
            plsc.addupdate_scatter(hist_v, [hi + r * 512], ones16)
        return 0

    lax.fori_loop(0, _CHUNK // 16, h1, 0)
    pltpu.sync_copy(hist_v, sp_hists.at[pl.ds(s * 1024, 1024)])
    plsc.subcore_barrier()

    @pl.when(s == 0)
    def _scan_level1():
        pltpu.sync_copy(sp_hists, big_v)

        def mbody(j, _):
            acc = big_v[pl.ds(j * 16, 16)]
            for w in range(1, 16):
                acc = acc + big_v[pl.ds(w * 1024 + j * 16, 16)]
            merged_v[pl.ds(j * 16, 16)] = acc
            return 0

        lax.fori_loop(0, 64, mbody, 0)
        for r in range(2):
            bstar, cab = _scan_hist(merged_v, r, kvec, iota16)
            ctrl_v[pl.ds(r * 16, 16)] = bstar
            misc_v[pl.ds(r * 16, 16)] = ones16 * (jnp.float32(_K) - cab)
        pltpu.sync_copy(ctrl_v, sp_ctrl)

    plsc.subcore_barrier()
    pltpu.sync_copy(sp_ctrl, ctrl_v)
    lax.fori_loop(0, 64, zbody, 0)

    def h2(i, _):
        for r in range(2):
            x = data_v[pl.ds(r * _CHUNK + i * 16, 16)]
            bits = plsc.bitcast(x, jnp.int32)
            hi = lax.shift_right_logical(bits, 22)
            b1 = ctrl_v[pl.ds(r * 16, 16)]
            mid = jnp.bitwise_and(lax.shift_right_logical(bits, 13), 511)
            plsc.addupdate_scatter(hist_v, [mid + r * 512], ones16,
                                   mask=hi == b1)
        return 0

    lax.fori_loop(0, _CHUNK // 16, h2, 0)
    pltpu.sync_copy(hist_v, sp_hists.at[pl.ds(s * 1024, 1024)])
    plsc.subcore_barrier()

    @pl.when(s == 0)
    def _scan_level2():
        pltpu.sync_copy(sp_hists, big_v)

        def mbody(j, _):
            acc = big_v[pl.ds(j * 16, 16)]
            for w in range(1, 16):
                acc = acc + big_v[pl.ds(w * 1024 + j * 16, 16)]
            merged_v[pl.ds(j * 16, 16)] = acc
            return 0

        lax.fori_loop(0, 64, mbody, 0)
        for r in range(2):
            remk2 = misc_v[pl.ds(r * 16, 16)]
            b1 = ctrl_v[pl.ds(r * 16, 16)]
            bstar2, _ = _scan_hist(merged_v, r, remk2, iota16)
            tbits = lax.shift_left(b1, 22) + lax.shift_left(bstar2, 13)
            outbuf_v[pl.ds(r * 16, 16)] = tbits
        pltpu.sync_copy(outbuf_v, out_hbm.at[pl.ds(c * 32, 32)])


def _bits_to_f32(bits_scalar):
    vecb = jnp.full((8, 128), bits_scalar, dtype=jnp.int32)
    vecf = lax.bitcast_convert_type(vecb, jnp.float32)
    r0 = lax.broadcasted_iota(jnp.int32, (8, 128), 0) == 0
    c0 = lax.broadcasted_iota(jnp.int32, (8, 128), 1) == 0
    return jnp.sum(jnp.where(jnp.logical_and(r0, c0), vecf, 0.0))


def _final_body(loss_ref, thr_ref, sums_ref, out_ref):
    kf = jnp.float32(_K)
    lbin_sum = jnp.float32(0.0)
    for b in range(_B):
        v = loss_ref[b]
        bits = lax.bitcast_convert_type(v, jnp.int32)
        tb = thr_ref[b * 16]
        cgt = jnp.sum((bits > tb).astype(jnp.float32))
        sgt = jnp.sum(jnp.where(bits > tb, v, 0.0))
        tval = _bits_to_f32(tb)
        lbin_sum = lbin_sum + sgt + (kf - cgt) * tval

    l_bin = lbin_sum / jnp.float32(_B * _K)
    l_inst = sums_ref[0, 0] / jnp.maximum(sums_ref[0, 1], 1.0)
    l_dir = sums_ref[0, 2] / jnp.maximum(sums_ref[0, 3], 1.0)
    out_ref[0, 0] = l_bin + l_inst + l_dir


def kernel(p_bin, p_inst, p_dir, binary_mask, instance_masks, direction_field):
    loss_map = pl.pallas_call(
        _bce_body,
        grid=(_B,),
        in_specs=[
            pl.BlockSpec((1, 1, _H, _W), lambda b: (b, 0, 0, 0)),
            pl.BlockSpec((1, 1, _H, _W), lambda b: (b, 0, 0, 0)),
        ],
        out_specs=pl.BlockSpec((1, _H, _W), lambda b: (b, 0, 0)),
        out_shape=jax.ShapeDtypeStruct((_B, _H, _W), jnp.float32),
        interpret=_INTERPRET,
    )(p_bin, binary_mask)

    sc_fn = functools.partial(
        pl.kernel,
        mesh=plsc.VectorSubcoreMesh(core_axis_name="c", subcore_axis_name="s"),
        out_type=jax.ShapeDtypeStruct((64,), jnp.int32),
        compiler_params=pltpu.CompilerParams(needs_layout_passes=False),
        scratch_types=[
            pltpu.VMEM((2 * _CHUNK,), jnp.float32),
            pltpu.VMEM((1024,), jnp.float32),
            pltpu.VMEM((16384,), jnp.float32),
            pltpu.VMEM((1024,), jnp.float32),
            pltpu.VMEM((32,), jnp.int32),
            pltpu.VMEM((32,), jnp.float32),
            pltpu.VMEM((32,), jnp.int32),
            pltpu.VMEM_SHARED((16384,), jnp.float32),
            pltpu.VMEM_SHARED((32,), jnp.int32),
        ],
    )(_sc_select)
    thr = sc_fn(loss_map.reshape(_B, _N))

    sums = pl.pallas_call(
        _sums_body,
        grid=(_B, _NH),
        in_specs=[
            pl.BlockSpec((1, 10, _BH, _W), lambda b, h: (b, 0, h, 0)),
            pl.BlockSpec((1, 2, _BH, _W), lambda b, h: (b, 0, h, 0)),
            pl.BlockSpec((1, 1, _BH, _W), lambda b, h: (b, 0, h, 0)),
            pl.BlockSpec((1, 10, _BH, _W), lambda b, h: (b, 0, h, 0)),
            pl.BlockSpec((1, 2, _BH, _W), lambda b, h: (b, 0, h, 0)),
        ],
        out_specs=pl.BlockSpec(memory_space=pltpu.SMEM),
        out_shape=jax.ShapeDtypeStruct((1, 4), jnp.float32),
        scratch_shapes=[pltpu.SMEM((4,), jnp.float32)],
        interpret=_INTERPRET,
    )(p_inst, p_dir, binary_mask, instance_masks, direction_field)

    total = pl.pallas_call(
        _final_body,
        in_specs=[
            pl.BlockSpec(memory_space=pltpu.VMEM),
            pl.BlockSpec(memory_space=pltpu.SMEM),
            pl.BlockSpec(memory_space=pltpu.SMEM),
        ],
        out_specs=pl.BlockSpec(memory_space=pltpu.SMEM),
        out_shape=jax.ShapeDtypeStruct((1, 1), jnp.float32),
        interpret=_INTERPRET,
    )(loss_map, thr, sums)

    return total[0, 0]


# speedup vs baseline: 24.0989x; 1.1256x over previous
import functools

import jax
import jax.numpy as jnp
from jax import lax
from jax.experimental import pallas as pl
from jax.experimental.pallas import tpu as pltpu
from jax.experimental.pallas import tpu_sc as plsc

_B, _H, _W = 4, 512, 512
_N = _H * _W
_K = max(1, int(_N * 0.1))
_BH = 128
_NH = _H // _BH

_CHUNK = _N // 16

_INTERPRET = False


def _bce_body(pbin_ref, bmask_ref, loss_ref):
    x = pbin_ref[0, 0]
    t = bmask_ref[0, 0]
    loss_ref[0] = (jnp.maximum(x, 0.0) - x * t
                   + jnp.log(1.0 + jnp.exp(-jnp.abs(x))))


def _sums_body(pinst_ref, pdir_ref, bmask_ref, imask_ref, dfield_ref,
               sums_ref, acc_ref):
    b = pl.program_id(0)
    h = pl.program_id(1)

    mx = pinst_ref[0, 0]
    for c in range(1, 10):
        mx = jnp.maximum(mx, pinst_ref[0, c])
    sexp = jnp.exp(pinst_ref[0, 0] - mx)
    for c in range(1, 10):
        sexp = sexp + jnp.exp(pinst_ref[0, c] - mx)
    lse = mx + jnp.log(sexp)
    best_m = imask_ref[0, 0]
    best_logit = pinst_ref[0, 0]
    for c in range(1, 10):
        m = imask_ref[0, c]
        upd = m > best_m
        best_m = jnp.where(upd, m, best_m)
        best_logit = jnp.where(upd, pinst_ref[0, c], best_logit)
    ce = lse - best_logit
    valid = (bmask_ref[0, 0] > 0.5).astype(jnp.float32)
    ce_part = jnp.sum(ce * valid)
    val_part = jnp.sum(valid)

    d0 = dfield_ref[0, 0]
    d1 = dfield_ref[0, 1]
    nrm = jnp.sqrt(d0 * d0 + d1 * d1)
    inv = 1.0 / jnp.maximum(nrm, 1e-6)
    dt0 = d0 * inv
    dt1 = d1 * inv
    dvalid = (nrm * inv > 1e-6).astype(jnp.float32)
    e0 = pdir_ref[0, 0] - dt0
    e1 = pdir_ref[0, 1] - dt1
    dir_part = jnp.sum((e0 * e0 + e1 * e1) * dvalid)
    dval_part = jnp.sum(dvalid)

    first = jnp.logical_and(b == 0, h == 0)

    @pl.when(first)
    def _init():
        acc_ref[0] = ce_part
        acc_ref[1] = val_part
        acc_ref[2] = dir_part
        acc_ref[3] = dval_part

    @pl.when(jnp.logical_not(first))
    def _acc():
        acc_ref[0] = acc_ref[0] + ce_part
        acc_ref[1] = acc_ref[1] + val_part
        acc_ref[2] = acc_ref[2] + dir_part
        acc_ref[3] = acc_ref[3] + dval_part

    last = jnp.logical_and(b == _B - 1, h == _NH - 1)

    @pl.when(last)
    def _emit():
        sums_ref[0, 0] = acc_ref[0]
        sums_ref[0, 1] = acc_ref[1]
        sums_ref[0, 2] = acc_ref[2]
        sums_ref[0, 3] = acc_ref[3]


def _scan_hist(merged_v, r, remk_vec, iota16):
    def sbody(i, carry):
        acc, cnt = carry
        v = 31 - i
        block = merged_v[pl.ds(r * 512 + v * 16, 16)]
        csum = plsc.cumsum(lax.rev(block, (0,))) + acc
        mask = csum >= remk_vec
        cnt = cnt + plsc.all_reduce_population_count(mask)
        acc = acc + jnp.sum(block)
        return acc, cnt

    _, cnt = lax.fori_loop(
        0, 32, sbody, (jnp.float32(0.0), jnp.zeros((16,), jnp.int32)))
    bstar = cnt - 1

    def abody(i, a):
        v = 31 - i
        block = merged_v[pl.ds(r * 512 + v * 16, 16)]
        binid = v * 16 + iota16
        return a + jnp.sum(jnp.where(binid > bstar, block, 0.0))

    cab = lax.fori_loop(0, 32, abody, jnp.float32(0.0))
    return bstar, cab


def _sc_select(loss_hbm, out_hbm, data_v, hist_v, big_v, merged_v,
               outbuf_v, sp_hists):
    c = lax.axis_index("c")
    s = lax.axis_index("s")
    ones16 = jnp.ones((16,), jnp.float32)
    zero16f = jnp.zeros((16,), jnp.float32)
    iota16 = lax.iota(jnp.int32, 16)
    kvec = ones16 * jnp.float32(_K)

    for r in range(2):
        pltpu.sync_copy(
            loss_hbm.at[c * 2 + r, pl.ds(s * _CHUNK, _CHUNK)],
            data_v.at[pl.ds(r * _CHUNK, _CHUNK)])

    def zbody(i, _):
        hist_v[pl.ds(i * 16, 16)] = zero16f
        return 0

    lax.fori_loop(0, 64, zbody, 0)

    def h1(i, _):
        for u in range(4):
            for r in range(2):
                x = data_v[pl.ds(r * _CHUNK + (i * 4 + u) * 16, 16)]
                bits = plsc.bitcast(x, jnp.int32)
                hi = lax.shift_right_logical(bits, 22)
                plsc.addupdate_scatter(hist_v, [hi + r * 512], ones16)
        return 0

    lax.fori_loop(0, _CHUNK // 64, h1, 0)
    pltpu.sync_copy(hist_v, sp_hists.at[pl.ds(s * 1024, 1024)])
    plsc.subcore_barrier()

    @pl.when(s == 0)
    def _scan_level1():
        pltpu.sync_copy(sp_hists, big_v)

        def mbody(j, _):
            acc = big_v[pl.ds(j * 16, 16)]
            for w in range(1, 16):
                acc = acc + big_v[pl.ds(w * 1024 + j * 16, 16)]
            merged_v[pl.ds(j * 16, 16)] = acc
            return 0

        lax.fori_loop(0, 64, mbody, 0)
        for r in range(2):
            bstar, cab = _scan_hist(merged_v, r, kvec, iota16)
            outbuf_v[pl.ds(r * 32, 16)] = lax.shift_left(bstar, 22)
            outbuf_v[pl.ds(r * 32 + 16, 16)] = cab.astype(jnp.int32) * \
                (iota16 * 0 + 1)
        pltpu.sync_copy(outbuf_v, out_hbm.at[pl.ds(c * 64, 64)])


def _bits_to_f32(bits_scalar):
    vecb = jnp.full((8, 128), bits_scalar, dtype=jnp.int32)
    vecf = lax.bitcast_convert_type(vecb, jnp.float32)
    r0 = lax.broadcasted_iota(jnp.int32, (8, 128), 0) == 0
    c0 = lax.broadcasted_iota(jnp.int32, (8, 128), 1) == 0
    return jnp.sum(jnp.where(jnp.logical_and(r0, c0), vecf, 0.0))


def _final_body(loss_ref, thr_ref, sums_ref, out_ref):
    kf = jnp.float32(_K)

    def off(b):
        return (b // 2) * 64 + (b % 2) * 32

    def bisect_pass(p, carry):
        prefix, remk, cab = carry
        step = lax.shift_left(jnp.int32(1), 21 - p)
        new_prefix, new_remk, new_cab = [], [], []
        for b in range(_B):
            bits = lax.bitcast_convert_type(loss_ref[b], jnp.int32)
            mid = prefix[b] + step
            c_mid = jnp.sum((bits >= mid).astype(jnp.float32))
            hi = c_mid - cab[b]
            go_hi = hi >= remk[b]
            new_prefix.append(jnp.where(go_hi, mid, prefix[b]))
            new_cab.append(jnp.where(go_hi, cab[b], c_mid))
            new_remk.append(jnp.where(go_hi, remk[b], remk[b] - hi))
        return tuple(new_prefix), tuple(new_remk), tuple(new_cab)

    prefix, _, _ = lax.fori_loop(
        0, 9, bisect_pass,
        (tuple(thr_ref[off(b)] for b in range(_B)),
         tuple(kf - thr_ref[off(b) + 16].astype(jnp.float32)
               for b in range(_B)),
         tuple(thr_ref[off(b) + 16].astype(jnp.float32) for b in range(_B))))

    lbin_sum = jnp.float32(0.0)
    for b in range(_B):
        v = loss_ref[b]
        bits = lax.bitcast_convert_type(v, jnp.int32)
        tb = prefix[b]
        cgt = jnp.sum((bits > tb).astype(jnp.float32))
        sgt = jnp.sum(jnp.where(bits > tb, v, 0.0))
        tval = _bits_to_f32(tb)
        lbin_sum = lbin_sum + sgt + (kf - cgt) * tval

    l_bin = lbin_sum / jnp.float32(_B * _K)
    l_inst = sums_ref[0, 0] / jnp.maximum(sums_ref[0, 1], 1.0)
    l_dir = sums_ref[0, 2] / jnp.maximum(sums_ref[0, 3], 1.0)
    out_ref[0, 0] = l_bin + l_inst + l_dir


def kernel(p_bin, p_inst, p_dir, binary_mask, instance_masks, direction_field):
    loss_map = pl.pallas_call(
        _bce_body,
        grid=(_B,),
        in_specs=[
            pl.BlockSpec((1, 1, _H, _W), lambda b: (b, 0, 0, 0)),
            pl.BlockSpec((1, 1, _H, _W), lambda b: (b, 0, 0, 0)),
        ],
        out_specs=pl.BlockSpec((1, _H, _W), lambda b: (b, 0, 0)),
        out_shape=jax.ShapeDtypeStruct((_B, _H, _W), jnp.float32),
        interpret=_INTERPRET,
    )(p_bin, binary_mask)

    sc_fn = functools.partial(
        pl.kernel,
        mesh=plsc.VectorSubcoreMesh(core_axis_name="c", subcore_axis_name="s"),
        out_type=jax.ShapeDtypeStruct((128,), jnp.int32),
        compiler_params=pltpu.CompilerParams(needs_layout_passes=False),
        scratch_types=[
            pltpu.VMEM((2 * _CHUNK,), jnp.float32),
            pltpu.VMEM((1024,), jnp.float32),
            pltpu.VMEM((16384,), jnp.float32),
            pltpu.VMEM((1024,), jnp.float32),
            pltpu.VMEM((64,), jnp.int32),
            pltpu.VMEM_SHARED((16384,), jnp.float32),
        ],
    )(_sc_select)
    thr = sc_fn(loss_map.reshape(_B, _N))

    sums = pl.pallas_call(
        _sums_body,
        grid=(_B, _NH),
        in_specs=[
            pl.BlockSpec((1, 10, _BH, _W), lambda b, h: (b, 0, h, 0)),
            pl.BlockSpec((1, 2, _BH, _W), lambda b, h: (b, 0, h, 0)),
            pl.BlockSpec((1, 1, _BH, _W), lambda b, h: (b, 0, h, 0)),
            pl.BlockSpec((1, 10, _BH, _W), lambda b, h: (b, 0, h, 0)),
            pl.BlockSpec((1, 2, _BH, _W), lambda b, h: (b, 0, h, 0)),
        ],
        out_specs=pl.BlockSpec(memory_space=pltpu.SMEM),
        out_shape=jax.ShapeDtypeStruct((1, 4), jnp.float32),
        scratch_shapes=[pltpu.SMEM((4,), jnp.float32)],
        interpret=_INTERPRET,
    )(p_inst, p_dir, binary_mask, instance_masks, direction_field)

    total = pl.pallas_call(
        _final_body,
        in_specs=[
            pl.BlockSpec(memory_space=pltpu.VMEM),
            pl.BlockSpec(memory_space=pltpu.SMEM),
            pl.BlockSpec(memory_space=pltpu.SMEM),
        ],
        out_specs=pl.BlockSpec(memory_space=pltpu.SMEM),
        out_shape=jax.ShapeDtypeStruct((1, 1), jnp.float32),
        interpret=_INTERPRET,
    )(loss_map, thr, sums)

    return total[0, 0]


# speedup vs baseline: 32.3775x; 1.3435x over previous
import jax
import jax.numpy as jnp
from jax import lax
from jax.experimental import pallas as pl
from jax.experimental.pallas import tpu as pltpu

_B, _H, _W = 4, 512, 512
_N = _H * _W
_K = max(1, int(_N * 0.1))
_BH = 256
_NH = _H // _BH
_NPASS = 8

_INTERPRET = False


def _fused_body(pbin_ref, pinst_ref, pdir_ref, bmask_ref, imask_ref,
                dfield_ref, loss_ref, sums_ref, acc_ref):
    b = pl.program_id(0)
    h = pl.program_id(1)

    x = pbin_ref[0, 0]
    t = bmask_ref[0, 0]
    bce = jnp.maximum(x, 0.0) - x * t + jnp.log(1.0 + jnp.exp(-jnp.abs(x)))
    loss_ref[0] = bce

    mx = pinst_ref[0, 0]
    for c in range(1, 10):
        mx = jnp.maximum(mx, pinst_ref[0, c])
    sexp = jnp.exp(pinst_ref[0, 0] - mx)
    for c in range(1, 10):
        sexp = sexp + jnp.exp(pinst_ref[0, c] - mx)
    lse = mx + jnp.log(sexp)
    best_m = imask_ref[0, 0]
    best_logit = pinst_ref[0, 0]
    for c in range(1, 10):
        m = imask_ref[0, c]
        upd = m > best_m
        best_m = jnp.where(upd, m, best_m)
        best_logit = jnp.where(upd, pinst_ref[0, c], best_logit)
    ce = lse - best_logit
    valid = (t > 0.5).astype(jnp.float32)
    ce_part = jnp.sum(ce * valid)
    val_part = jnp.sum(valid)

    d0 = dfield_ref[0, 0]
    d1 = dfield_ref[0, 1]
    nrm = jnp.sqrt(d0 * d0 + d1 * d1)
    inv = 1.0 / jnp.maximum(nrm, 1e-6)
    dt0 = d0 * inv
    dt1 = d1 * inv
    dvalid = (nrm * inv > 1e-6).astype(jnp.float32)
    e0 = pdir_ref[0, 0] - dt0
    e1 = pdir_ref[0, 1] - dt1
    dir_part = jnp.sum((e0 * e0 + e1 * e1) * dvalid)
    dval_part = jnp.sum(dvalid)

    first = jnp.logical_and(b == 0, h == 0)

    @pl.when(first)
    def _init():
        acc_ref[0] = ce_part
        acc_ref[1] = val_part
        acc_ref[2] = dir_part
        acc_ref[3] = dval_part

    @pl.when(jnp.logical_not(first))
    def _acc():
        acc_ref[0] = acc_ref[0] + ce_part
        acc_ref[1] = acc_ref[1] + val_part
        acc_ref[2] = acc_ref[2] + dir_part
        acc_ref[3] = acc_ref[3] + dval_part

    last = jnp.logical_and(b == _B - 1, h == _NH - 1)

    @pl.when(last)
    def _emit():
        sums_ref[0, 0] = acc_ref[0]
        sums_ref[0, 1] = acc_ref[1]
        sums_ref[0, 2] = acc_ref[2]
        sums_ref[0, 3] = acc_ref[3]


def _bits_to_f32(bits_scalar):
    vecb = jnp.full((8, 128), bits_scalar, dtype=jnp.int32)
    vecf = lax.bitcast_convert_type(vecb, jnp.float32)
    r0 = lax.broadcasted_iota(jnp.int32, (8, 128), 0) == 0
    c0 = lax.broadcasted_iota(jnp.int32, (8, 128), 1) == 0
    return jnp.sum(jnp.where(jnp.logical_and(r0, c0), vecf, 0.0))


def _select_body(loss_ref, sums_ref, out_ref):
    kf = jnp.float32(_K)

    def bisect_pass(p, carry):
        prefix, remk, cab = carry
        step = lax.shift_left(jnp.int32(1), 30 - p)
        new_prefix, new_remk, new_cab = [], [], []
        for b in range(_B):
            bits = lax.bitcast_convert_type(loss_ref[b], jnp.int32)
            mid = prefix[b] + step
            c_mid = jnp.sum((bits >= mid).astype(jnp.float32))
            hi = c_mid - cab[b]
            go_hi = hi >= remk[b]
            new_prefix.append(jnp.where(go_hi, mid, prefix[b]))
            new_cab.append(jnp.where(go_hi, cab[b], c_mid))
            new_remk.append(jnp.where(go_hi, remk[b], remk[b] - hi))
        return tuple(new_prefix), tuple(new_remk), tuple(new_cab)

    prefix, _, _ = lax.fori_loop(
        0, 20, bisect_pass,
        (tuple(jnp.int32(0) for _ in range(_B)),
         tuple(kf for _ in range(_B)),
         tuple(jnp.float32(0.0) for _ in range(_B))))

    lbin_sum = jnp.float32(0.0)
    for b in range(_B):
        v = loss_ref[b]
        bits = lax.bitcast_convert_type(v, jnp.int32)
        cgt = jnp.sum((bits > prefix[b]).astype(jnp.float32))
        sgt = jnp.sum(jnp.where(bits > prefix[b], v, 0.0))
        tval = _bits_to_f32(prefix[b])
        lbin_sum = lbin_sum + sgt + (kf - cgt) * tval

    l_bin = lbin_sum / jnp.float32(_B * _K)
    l_inst = sums_ref[0, 0] / jnp.maximum(sums_ref[0, 1], 1.0)
    l_dir = sums_ref[0, 2] / jnp.maximum(sums_ref[0, 3], 1.0)
    out_ref[0, 0] = l_bin + l_inst + l_dir


def kernel(p_bin, p_inst, p_dir, binary_mask, instance_masks, direction_field):
    loss_map, sums = pl.pallas_call(
        _fused_body,
        grid=(_B, _NH),
        in_specs=[
            pl.BlockSpec((1, 1, _BH, _W), lambda b, h: (b, 0, h, 0)),
            pl.BlockSpec((1, 10, _BH, _W), lambda b, h: (b, 0, h, 0)),
            pl.BlockSpec((1, 2, _BH, _W), lambda b, h: (b, 0, h, 0)),
            pl.BlockSpec((1, 1, _BH, _W), lambda b, h: (b, 0, h, 0)),
            pl.BlockSpec((1, 10, _BH, _W), lambda b, h: (b, 0, h, 0)),
            pl.BlockSpec((1, 2, _BH, _W), lambda b, h: (b, 0, h, 0)),
        ],
        out_specs=[
            pl.BlockSpec((1, _BH, _W), lambda b, h: (b, h, 0)),
            pl.BlockSpec(memory_space=pltpu.SMEM),
        ],
        out_shape=[
            jax.ShapeDtypeStruct((_B, _H, _W), jnp.float32),
            jax.ShapeDtypeStruct((1, 4), jnp.float32),
        ],
        scratch_shapes=[pltpu.SMEM((4,), jnp.float32)],
        interpret=_INTERPRET,
    )(p_bin, p_inst, p_dir, binary_mask, instance_masks, direction_field)

    total = pl.pallas_call(
        _select_body,
        in_specs=[
            pl.BlockSpec(memory_space=pltpu.VMEM),
            pl.BlockSpec(memory_space=pltpu.SMEM),
        ],
        out_specs=pl.BlockSpec(memory_space=pltpu.SMEM),
        out_shape=jax.ShapeDtypeStruct((1, 1), jnp.float32),
        interpret=_INTERPRET,
    )(loss_map, sums)

    return total[0, 0]


# speedup vs baseline: 32.3960x; 1.0006x over previous
import jax
import jax.numpy as jnp
from jax import lax
from jax.experimental import pallas as pl
from jax.experimental.pallas import tpu as pltpu

_B, _H, _W = 4, 512, 512
_N = _H * _W
_K = max(1, int(_N * 0.1))
_BH = 256
_NH = _H // _BH

_INTERPRET = False


def _fused_body(pbin_ref, pinst_ref, pdir_ref, bmask_ref, imask_ref,
                dfield_ref, loss_ref, sums_ref, acc_ref):
    b = pl.program_id(0)
    h = pl.program_id(1)

    x = pbin_ref[0, 0]
    t = bmask_ref[0, 0]
    bce = jnp.maximum(x, 0.0) - x * t + jnp.log(1.0 + jnp.exp(-jnp.abs(x)))
    loss_ref[0] = bce

    mx = pinst_ref[0, 0]
    for c in range(1, 10):
        mx = jnp.maximum(mx, pinst_ref[0, c])
    sexp = jnp.exp(pinst_ref[0, 0] - mx)
    for c in range(1, 10):
        sexp = sexp + jnp.exp(pinst_ref[0, c] - mx)
    lse = mx + jnp.log(sexp)
    best_m = imask_ref[0, 0]
    best_logit = pinst_ref[0, 0]
    for c in range(1, 10):
        m = imask_ref[0, c]
        upd = m > best_m
        best_m = jnp.where(upd, m, best_m)
        best_logit = jnp.where(upd, pinst_ref[0, c], best_logit)
    ce = lse - best_logit
    valid = (t > 0.5).astype(jnp.float32)
    ce_part = jnp.sum(ce * valid)
    val_part = jnp.sum(valid)

    d0 = dfield_ref[0, 0]
    d1 = dfield_ref[0, 1]
    nrm = jnp.sqrt(d0 * d0 + d1 * d1)
    inv = 1.0 / jnp.maximum(nrm, 1e-6)
    dt0 = d0 * inv
    dt1 = d1 * inv
    dvalid = (nrm * inv > 1e-6).astype(jnp.float32)
    e0 = pdir_ref[0, 0] - dt0
    e1 = pdir_ref[0, 1] - dt1
    dir_part = jnp.sum((e0 * e0 + e1 * e1) * dvalid)
    dval_part = jnp.sum(dvalid)

    first = jnp.logical_and(b == 0, h == 0)

    @pl.when(first)
    def _init():
        acc_ref[0] = ce_part
        acc_ref[1] = val_part
        acc_ref[2] = dir_part
        acc_ref[3] = dval_part

    @pl.when(jnp.logical_not(first))
    def _acc():
        acc_ref[0] = acc_ref[0] + ce_part
        acc_ref[1] = acc_ref[1] + val_part
        acc_ref[2] = acc_ref[2] + dir_part
        acc_ref[3] = acc_ref[3] + dval_part

    last = jnp.logical_and(b == _B - 1, h == _NH - 1)

    @pl.when(last)
    def _emit():
        sums_ref[0, 0] = acc_ref[0]
        sums_ref[0, 1] = acc_ref[1]
        sums_ref[0, 2] = acc_ref[2]
        sums_ref[0, 3] = acc_ref[3]


def _bits_to_f32(bits_scalar):
    vecb = jnp.full((8, 128), bits_scalar, dtype=jnp.int32)
    vecf = lax.bitcast_convert_type(vecb, jnp.float32)
    r0 = lax.broadcasted_iota(jnp.int32, (8, 128), 0) == 0
    c0 = lax.broadcasted_iota(jnp.int32, (8, 128), 1) == 0
    return jnp.sum(jnp.where(jnp.logical_and(r0, c0), vecf, 0.0))


def _select_body(loss_ref, sums_ref, out_ref):
    kf = jnp.float32(_K)

    def bisect_pass(p, carry):
        prefix, remk, cab = carry
        step = lax.shift_left(jnp.int32(1), 30 - p)
        new_prefix, new_remk, new_cab = [], [], []
        for b in range(_B):
            bits = lax.bitcast_convert_type(loss_ref[b], jnp.int32)
            mid = prefix[b] + step
            c_mid = jnp.sum((bits >= mid).astype(jnp.float32))
            hi = c_mid - cab[b]
            go_hi = hi >= remk[b]
            new_prefix.append(jnp.where(go_hi, mid, prefix[b]))
            new_cab.append(jnp.where(go_hi, cab[b], c_mid))
            new_remk.append(jnp.where(go_hi, remk[b], remk[b] - hi))
        return tuple(new_prefix), tuple(new_remk), tuple(new_cab)

    prefix, _, _ = lax.fori_loop(
        0, 20, bisect_pass,
        (tuple(jnp.int32(0) for _ in range(_B)),
         tuple(kf for _ in range(_B)),
         tuple(jnp.float32(0.0) for _ in range(_B))))

    lbin_sum = jnp.float32(0.0)
    for b in range(_B):
        v = loss_ref[b]
        bits = lax.bitcast_convert_type(v, jnp.int32)
        cgt = jnp.sum((bits > prefix[b]).astype(jnp.float32))
        sgt = jnp.sum(jnp.where(bits > prefix[b], v, 0.0))
        tval = _bits_to_f32(prefix[b])
        lbin_sum = lbin_sum + sgt + (kf - cgt) * tval

    l_bin = lbin_sum / jnp.float32(_B * _K)
    l_inst = sums_ref[0, 0] / jnp.maximum(sums_ref[0, 1], 1.0)
    l_dir = sums_ref[0, 2] / jnp.maximum(sums_ref[0, 3], 1.0)
    out_ref[0, 0] = l_bin + l_inst + l_dir


def kernel(p_bin, p_inst, p_dir, binary_mask, instance_masks, direction_field):
    loss_map, sums = pl.pallas_call(
        _fused_body,
        grid=(_B, _NH),
        in_specs=[
            pl.BlockSpec((1, 1, _BH, _W), lambda b, h: (b, 0, h, 0)),
            pl.BlockSpec((1, 10, _BH, _W), lambda b, h: (b, 0, h, 0)),
            pl.BlockSpec((1, 2, _BH, _W), lambda b, h: (b, 0, h, 0)),
            pl.BlockSpec((1, 1, _BH, _W), lambda b, h: (b, 0, h, 0)),
            pl.BlockSpec((1, 10, _BH, _W), lambda b, h: (b, 0, h, 0)),
            pl.BlockSpec((1, 2, _BH, _W), lambda b, h: (b, 0, h, 0)),
        ],
        out_specs=[
            pl.BlockSpec((1, _BH, _W), lambda b, h: (b, h, 0)),
            pl.BlockSpec(memory_space=pltpu.SMEM),
        ],
        out_shape=[
            jax.ShapeDtypeStruct((_B, _H, _W), jnp.float32),
            jax.ShapeDtypeStruct((1, 4), jnp.float32),
        ],
        scratch_shapes=[pltpu.SMEM((4,), jnp.float32)],
        interpret=_INTERPRET,
    )(p_bin, p_inst, p_dir, binary_mask, instance_masks, direction_field)

    total = pl.pallas_call(
        _select_body,
        in_specs=[
            pl.BlockSpec(memory_space=pltpu.VMEM),
            pl.BlockSpec(memory_space=pltpu.SMEM),
        ],
        out_specs=pl.BlockSpec(memory_space=pltpu.SMEM),
        out_shape=jax.ShapeDtypeStruct((1, 1), jnp.float32),
        interpret=_INTERPRET,
    )(loss_map, sums)

    return total[0, 0]
